# baseline (speedup 1.0000x reference)
import functools

import jax
import jax.numpy as jnp
from jax import lax
from jax.experimental import pallas as pl
from jax.experimental.pallas import tpu as pltpu
from jax.experimental.pallas import tpu_sc as plsc

N = 10000
E = 160000
S = 2000
G = 64
NW = 32
PW = 80

_MESH = dict(core_axis_name="c", subcore_axis_name="s")


def _sc_gather(table, idx, m_in):
    per_w = E // NW
    chunk = 200 if m_in > 64 else 1000

    mesh = plsc.VectorSubcoreMesh(**_MESH)

    @functools.partial(
        pl.kernel, mesh=mesh,
        compiler_params=pltpu.CompilerParams(use_tc_tiling_on_sc=False, needs_layout_passes=False),
        out_type=jax.ShapeDtypeStruct((E, m_in), jnp.float32),
        scratch_types=[
            pltpu.VMEM((chunk,), jnp.int32),
            pltpu.VMEM((chunk, m_in), jnp.float32),
            pltpu.SemaphoreType.DMA,
        ],
    )
    def k(table_hbm, idx_hbm, out_hbm, idx_v, rows_v, sem):
        wid = lax.axis_index("s") * 2 + lax.axis_index("c")
        base = wid * per_w

        def body(i, carry):
            off = base + i * chunk
            pltpu.sync_copy(idx_hbm.at[pl.ds(off, chunk)], idx_v)
            pltpu.async_copy(table_hbm.at[idx_v], rows_v, sem).wait()
            pltpu.sync_copy(rows_v, out_hbm.at[pl.ds(off, chunk)])
            return carry

        lax.fori_loop(0, per_w // chunk, body, 0)

    return k(table, idx)


def _sc_scatter(msg, dst, m_out):
    per_w = E // NW
    chunk = 1000
    n_pad = 10240
    rows_s = n_pad // 16
    cw = 32
    phases = m_out // cw

    mesh = plsc.VectorSubcoreMesh(**_MESH)

    @functools.partial(
        pl.kernel, mesh=mesh,
        compiler_params=pltpu.CompilerParams(use_tc_tiling_on_sc=False, needs_layout_passes=False),
        out_type=jax.ShapeDtypeStruct((2, phases, n_pad, cw), jnp.float32),
        scratch_types=[
            pltpu.VMEM((chunk,), jnp.int32),
            pltpu.VMEM((chunk, cw), jnp.float32),
            pltpu.VMEM((rows_s, cw), jnp.float32),
            pltpu.VMEM_SHARED((n_pad, cw), jnp.float32),
        ],
    )
    def k(msg_hbm, dst_hbm, out_hbm, idx_v, buf_v, stage_v, aggr_sh):
        cid = lax.axis_index("c")
        sid = lax.axis_index("s")
        wid = sid * 2 + cid
        base = wid * per_w
        zz = jnp.zeros((16,), jnp.float32)

        def zrow(r, carry):
            for cc in range(cw // 16):
                stage_v[r, pl.ds(cc * 16, 16)] = zz
            return carry

        lax.fori_loop(0, rows_s, zrow, 0)

        for p in range(phases):
            pltpu.sync_copy(stage_v, aggr_sh.at[pl.ds(sid * rows_s, rows_s)])
            plsc.subcore_barrier()

            def body(i, carry):
                off = base + i * chunk
                pltpu.sync_copy(dst_hbm.at[pl.ds(off, chunk)], idx_v)
                pltpu.sync_copy(msg_hbm.at[p, pl.ds(off, chunk)], buf_v)
                pltpu.sync_copy(buf_v, aggr_sh.at[idx_v], add=True)
                return carry

            lax.fori_loop(0, per_w // chunk, body, 0)
            plsc.subcore_barrier()

            pltpu.sync_copy(aggr_sh.at[pl.ds(sid * rows_s, rows_s)], stage_v)
            pltpu.sync_copy(
                stage_v,
                out_hbm.at[cid, p, pl.ds(sid * rows_s, rows_s)])
            if p + 1 < phases:
                plsc.subcore_barrier()
                lax.fori_loop(0, rows_s, zrow, 0)

    return k(msg, dst)


def _tc_msg(ea_p, x_src, w1p, b1, w2, b2, m_in, m_out):
    blk = 256
    grid = E // blk

    def body(ea_ref, xs_ref, w1_ref, b1_ref, w2_ref, b2_ref, out_ref):
        h = jnp.maximum(
            jnp.dot(ea_ref[...], w1_ref[...],
                    preferred_element_type=jnp.float32) + b1_ref[...], 0.0)
        w = jnp.dot(h, w2_ref[...],
                    preferred_element_type=jnp.float32) + b2_ref[...]
        w3 = w.reshape(blk, m_in, m_out)
        msg = jnp.sum(w3 * xs_ref[...][:, :, None], axis=1)
        for p in range(m_out // 32):
            out_ref[p, :, :] = msg[:, p * 32:(p + 1) * 32]

    return pl.pallas_call(
        body,
        grid=(grid,),
        in_specs=[
            pl.BlockSpec((blk, 8), lambda i: (i, 0)),
            pl.BlockSpec((blk, m_in), lambda i: (i, 0)),
            pl.BlockSpec((8, 128), lambda i: (0, 0)),
            pl.BlockSpec((1, 128), lambda i: (0, 0)),
            pl.BlockSpec((128, m_in * m_out), lambda i: (0, 0)),
            pl.BlockSpec((1, m_in * m_out), lambda i: (0, 0)),
        ],
        out_specs=pl.BlockSpec((m_out // 32, blk, 32), lambda i: (0, i, 0)),
        out_shape=jax.ShapeDtypeStruct((m_out // 32, E, 32), jnp.float32),
        compiler_params=pltpu.CompilerParams(
            dimension_semantics=("arbitrary",)),
    )(ea_p, x_src, w1p, b1, w2, b2)


def _tc_node(x, root, a0, a1, bias, m_in, m_out):
    blk = 400
    grid = N // blk

    def body(x_ref, r_ref, a0_ref, a1_ref, b_ref, out_ref):
        v = (jnp.dot(x_ref[...], r_ref[...],
                     preferred_element_type=jnp.float32)
             + a0_ref[...] + a1_ref[...] + b_ref[...])
        out_ref[...] = jnp.where(v > 0, v, jnp.exp(jnp.minimum(v, 0.0)) - 1.0)

    return pl.pallas_call(
        body,
        grid=(grid,),
        in_specs=[
            pl.BlockSpec((blk, m_in), lambda i: (i, 0)),
            pl.BlockSpec((m_in, m_out), lambda i: (0, 0)),
            pl.BlockSpec((blk, m_out), lambda i: (i, 0)),
            pl.BlockSpec((blk, m_out), lambda i: (i, 0)),
            pl.BlockSpec((1, m_out), lambda i: (0, 0)),
        ],
        out_specs=pl.BlockSpec((blk, m_out), lambda i: (i, 0)),
        out_shape=jax.ShapeDtypeStruct((N, m_out), jnp.float32),
        compiler_params=pltpu.CompilerParams(
            dimension_semantics=("arbitrary",)),
    )(x, root, a0, a1, bias)


def _sc_pool(x3, nts, stg_p):
    n_active = 25
    rows_w = S // n_active

    mesh = plsc.VectorSubcoreMesh(**_MESH)

    @functools.partial(
        pl.kernel, mesh=mesh,
        compiler_params=pltpu.CompilerParams(use_tc_tiling_on_sc=False, needs_layout_passes=False),
        out_type=jax.ShapeDtypeStruct((2, G, PW), jnp.float32),
        scratch_types=[
            pltpu.VMEM((N,), jnp.int32),
            pltpu.VMEM((rows_w,), jnp.int32),
            pltpu.VMEM((rows_w, 64), jnp.float32),
            pltpu.VMEM((rows_w, PW), jnp.float32),
            pltpu.VMEM((rows_w,), jnp.int32),
            pltpu.VMEM((G, PW), jnp.float32),
            pltpu.VMEM_SHARED((G, PW), jnp.float32),
            pltpu.SemaphoreType.DMA,
        ],
    )
    def k(x3_hbm, nts_hbm, stg_hbm, out_hbm,
          nts_v, cidx_v, rows_v, prows_v, stg_v, gbuf_v, gsum_sh, sem):
        cid = lax.axis_index("c")
        sid = lax.axis_index("s")
        wid = sid * 2 + cid

        zz = jnp.zeros((16,), jnp.float32)
        for r in range(G):
            for cc in range(PW // 16):
                gbuf_v[r, pl.ds(cc * 16, 16)] = zz

        @pl.when(sid == 0)
        def _():
            pltpu.sync_copy(gbuf_v, gsum_sh)

        plsc.subcore_barrier()

        @pl.when(wid < n_active)
        def _():
            pltpu.sync_copy(nts_hbm, nts_v)
            base = wid * rows_w
            lane = lax.iota(jnp.int32, 16)
            for j in range(rows_w // 16):
                s = base + j * 16 + lane
                lo = jnp.full((16,), -1, jnp.int32)
                hi = jnp.full((16,), N, jnp.int32)
                for _step in range(14):
                    mid = jnp.maximum((lo + hi) >> 1, 0)
                    v = plsc.load_gather(nts_v, [mid])
                    pred = v < s
                    lo = jnp.where(pred, mid, lo)
                    hi = jnp.where(pred, hi, mid)
                cidx_v[pl.ds(j * 16, 16)] = jnp.minimum(hi, N - 1)
            pltpu.async_copy(x3_hbm.at[cidx_v], rows_v, sem).wait()
            e0 = jnp.where(lane == 0, 1.0, 0.0).astype(jnp.float32)
            for j in range(rows_w):
                for cc in range(4):
                    prows_v[j, pl.ds(cc * 16, 16)] = rows_v[j, pl.ds(cc * 16, 16)]
                prows_v[j, pl.ds(64, 16)] = e0
            pltpu.sync_copy(stg_hbm.at[pl.ds(base, rows_w)], stg_v)
            pltpu.sync_copy(prows_v, gsum_sh.at[stg_v], add=True)

        plsc.subcore_barrier()

        @pl.when(sid == 0)
        def _():
            pltpu.sync_copy(gsum_sh, gbuf_v)
            pltpu.sync_copy(gbuf_v, out_hbm.at[cid])

    return k(x3, nts, stg_p)


def _tc_mlp(p0, p1, fc1_w, fc1_b, fc2_w, fc2_b, fc3_w, fc3_b):
    def body(p0_ref, p1_ref, w1_ref, b1_ref, w2_ref, b2_ref, w3_ref, b3_ref,
             out_ref):
        ps = p0_ref[...] + p1_ref[...]
        cnt = jnp.maximum(ps[:, 64:65], 1.0)
        xg = ps[:, 0:64] / cnt
        v = jnp.dot(xg, w1_ref[...], preferred_element_type=jnp.float32) \
            + b1_ref[...]
        v = jnp.where(v > 0, v, jnp.exp(jnp.minimum(v, 0.0)) - 1.0)
        v = jnp.dot(v, w2_ref[...], preferred_element_type=jnp.float32) \
            + b2_ref[...]
        v = jnp.where(v > 0, v, jnp.exp(jnp.minimum(v, 0.0)) - 1.0)
        out_ref[...] = jnp.dot(v, w3_ref[...],
                               preferred_element_type=jnp.float32) + b3_ref[...]

    return pl.pallas_call(
        body,
        out_shape=jax.ShapeDtypeStruct((G, 1), jnp.float32),
    )(p0, p1, fc1_w, fc1_b.reshape(1, -1), fc2_w, fc2_b.reshape(1, -1),
      fc3_w, fc3_b.reshape(1, -1))


def kernel(x, edge_index, edge_attr, node_to_subgraph, subgraph_to_graph,
           W1_1, b1_1, W2_1, b2_1, root_1, bias_1,
           W1_2, b1_2, W2_2, b2_2, root_2, bias_2,
           W1_3, b1_3, W2_3, b2_3, root_3, bias_3,
           fc1_w, fc1_b, fc2_w, fc2_b, fc3_w, fc3_b):
    src = edge_index[0]
    dst = edge_index[1]
    ea_p = jnp.pad(edge_attr, ((0, 0), (0, 3)))

    conv = [
        (W1_1, b1_1, W2_1, b2_1, root_1, bias_1, 128, 32),
        (W1_2, b1_2, W2_2, b2_2, root_2, bias_2, 32, 64),
        (W1_3, b1_3, W2_3, b2_3, root_3, bias_3, 64, 64),
    ]
    for (w1, b1, w2, b2, root, bias, m_in, m_out) in conv:
        w1p = jnp.pad(w1, ((0, 3), (0, 0)))
        x_src = _sc_gather(x, src, m_in)
        msg = _tc_msg(ea_p, x_src, w1p, b1.reshape(1, -1), w2,
                      b2.reshape(1, -1), m_in, m_out)
        aggr_p = _sc_scatter(msg, dst, m_out)
        aggr0 = jnp.concatenate([aggr_p[0, p] for p in range(m_out // 32)],
                                axis=1)
        aggr1 = jnp.concatenate([aggr_p[1, p] for p in range(m_out // 32)],
                                axis=1)
        x = _tc_node(x, root, aggr0, aggr1, bias.reshape(1, -1),
                     m_in, m_out)

    pool_p = _sc_pool(x, node_to_subgraph.astype(jnp.int32),
                      subgraph_to_graph.astype(jnp.int32))
    out = _tc_mlp(pool_p[0], pool_p[1], fc1_w, fc1_b, fc2_w, fc2_b,
                  fc3_w, fc3_b)
    return out.reshape(-1)


# speedup vs baseline: 3.0188x; 3.0188x over previous
import functools

import jax
import jax.numpy as jnp
from jax import lax
from jax.experimental import pallas as pl
from jax.experimental.pallas import tpu as pltpu
from jax.experimental.pallas import tpu_sc as plsc

N = 10000
E = 160000
S = 2000
G = 64
NW = 32
PW = 80

_MESH = dict(core_axis_name="c", subcore_axis_name="s")


def _sc_gather(table, idx, m_in):
    per_w = E // NW
    chunk = 200 if m_in > 64 else 1000

    mesh = plsc.VectorSubcoreMesh(**_MESH)

    @functools.partial(
        pl.kernel, mesh=mesh,
        compiler_params=pltpu.CompilerParams(use_tc_tiling_on_sc=False, needs_layout_passes=False),
        out_type=jax.ShapeDtypeStruct((E, m_in), jnp.float32),
        scratch_types=[
            pltpu.VMEM((chunk,), jnp.int32),
            pltpu.VMEM((chunk, m_in), jnp.float32),
            pltpu.SemaphoreType.DMA,
        ],
    )
    def k(table_hbm, idx_hbm, out_hbm, idx_v, rows_v, sem):
        wid = lax.axis_index("s") * 2 + lax.axis_index("c")
        base = wid * per_w

        def body(i, carry):
            off = base + i * chunk
            pltpu.sync_copy(idx_hbm.at[pl.ds(off, chunk)], idx_v)
            pltpu.async_copy(table_hbm.at[idx_v], rows_v, sem).wait()
            pltpu.sync_copy(rows_v, out_hbm.at[pl.ds(off, chunk)])
            return carry

        lax.fori_loop(0, per_w // chunk, body, 0)

    return k(table, idx)


def _sc_scatter(msg, dst, m_out):
    per_w = E // NW
    chunk = 1000
    n_pad = 10240
    rows_s = n_pad // 16
    cw = 32
    phases = m_out // cw

    mesh = plsc.VectorSubcoreMesh(**_MESH)

    @functools.partial(
        pl.kernel, mesh=mesh,
        compiler_params=pltpu.CompilerParams(use_tc_tiling_on_sc=False, needs_layout_passes=False),
        out_type=jax.ShapeDtypeStruct((2, phases, n_pad, cw), jnp.float32),
        scratch_types=[
            pltpu.VMEM((chunk,), jnp.int32),
            pltpu.VMEM((chunk, cw), jnp.float32),
            pltpu.VMEM((rows_s, cw), jnp.float32),
            pltpu.VMEM_SHARED((n_pad, cw), jnp.float32),
        ],
    )
    def k(msg_hbm, dst_hbm, out_hbm, idx_v, buf_v, stage_v, aggr_sh):
        cid = lax.axis_index("c")
        sid = lax.axis_index("s")
        wid = sid * 2 + cid
        base = wid * per_w
        zz = jnp.zeros((16,), jnp.float32)

        def zrow(r, carry):
            for cc in range(cw // 16):
                stage_v[r, pl.ds(cc * 16, 16)] = zz
            return carry

        lax.fori_loop(0, rows_s, zrow, 0)

        for p in range(phases):
            pltpu.sync_copy(stage_v, aggr_sh.at[pl.ds(sid * rows_s, rows_s)])
            plsc.subcore_barrier()

            def body(i, carry):
                off = base + i * chunk
                pltpu.sync_copy(dst_hbm.at[pl.ds(off, chunk)], idx_v)
                pltpu.sync_copy(msg_hbm.at[p, pl.ds(off, chunk)], buf_v)
                pltpu.sync_copy(buf_v, aggr_sh.at[idx_v], add=True)
                return carry

            lax.fori_loop(0, per_w // chunk, body, 0)
            plsc.subcore_barrier()

            pltpu.sync_copy(aggr_sh.at[pl.ds(sid * rows_s, rows_s)], stage_v)
            pltpu.sync_copy(
                stage_v,
                out_hbm.at[cid, p, pl.ds(sid * rows_s, rows_s)])
            if p + 1 < phases:
                plsc.subcore_barrier()
                lax.fori_loop(0, rows_s, zrow, 0)

    return k(msg, dst)


def _tc_msg(ea_p, x_src, w1p, b1, w2t, b2t, red, m_in, m_out):
    blk = 256
    grid = E // blk
    cols = m_in * m_out
    rep = 128 // m_in

    def body(ea_ref, xs_ref, w1_ref, b1_ref, w2_ref, b2_ref, red_ref, out_ref):
        h = jnp.maximum(
            jnp.dot(ea_ref[...], w1_ref[...],
                    preferred_element_type=jnp.float32) + b1_ref[...], 0.0)
        w = jnp.dot(h.astype(jnp.bfloat16), w2_ref[...],
                    preferred_element_type=jnp.float32) + b2_ref[...]
        xs = xs_ref[...]
        xt128 = jnp.concatenate([xs] * rep, axis=1) if rep > 1 else xs
        xt = jnp.concatenate([xt128] * (cols // 128), axis=1)
        p = w * xt
        msg = jnp.dot(p, red_ref[...], preferred_element_type=jnp.float32)
        for q in range(m_out // 32):
            out_ref[q, :, :] = msg[:, q * 32:(q + 1) * 32]

    return pl.pallas_call(
        body,
        grid=(grid,),
        in_specs=[
            pl.BlockSpec((blk, 8), lambda i: (i, 0)),
            pl.BlockSpec((blk, m_in), lambda i: (i, 0)),
            pl.BlockSpec((8, 128), lambda i: (0, 0)),
            pl.BlockSpec((1, 128), lambda i: (0, 0)),
            pl.BlockSpec((128, cols), lambda i: (0, 0)),
            pl.BlockSpec((1, cols), lambda i: (0, 0)),
            pl.BlockSpec((cols, m_out), lambda i: (0, 0)),
        ],
        out_specs=pl.BlockSpec((m_out // 32, blk, 32), lambda i: (0, i, 0)),
        out_shape=jax.ShapeDtypeStruct((m_out // 32, E, 32), jnp.float32),
        compiler_params=pltpu.CompilerParams(
            dimension_semantics=("arbitrary",)),
    )(ea_p, x_src, w1p, b1, w2t, b2t, red)


def _tc_node(x, root, a0, a1, bias, m_in, m_out):
    blk = 400
    grid = N // blk

    def body(x_ref, r_ref, a0_ref, a1_ref, b_ref, out_ref):
        v = (jnp.dot(x_ref[...], r_ref[...],
                     preferred_element_type=jnp.float32)
             + a0_ref[...] + a1_ref[...] + b_ref[...])
        out_ref[...] = jnp.where(v > 0, v, jnp.exp(jnp.minimum(v, 0.0)) - 1.0)

    return pl.pallas_call(
        body,
        grid=(grid,),
        in_specs=[
            pl.BlockSpec((blk, m_in), lambda i: (i, 0)),
            pl.BlockSpec((m_in, m_out), lambda i: (0, 0)),
            pl.BlockSpec((blk, m_out), lambda i: (i, 0)),
            pl.BlockSpec((blk, m_out), lambda i: (i, 0)),
            pl.BlockSpec((1, m_out), lambda i: (0, 0)),
        ],
        out_specs=pl.BlockSpec((blk, m_out), lambda i: (i, 0)),
        out_shape=jax.ShapeDtypeStruct((N, m_out), jnp.float32),
        compiler_params=pltpu.CompilerParams(
            dimension_semantics=("arbitrary",)),
    )(x, root, a0, a1, bias)


def _sc_pool(x3, nts, stg_p):
    n_active = 25
    rows_w = S // n_active

    mesh = plsc.VectorSubcoreMesh(**_MESH)

    @functools.partial(
        pl.kernel, mesh=mesh,
        compiler_params=pltpu.CompilerParams(use_tc_tiling_on_sc=False, needs_layout_passes=False),
        out_type=jax.ShapeDtypeStruct((2, G, PW), jnp.float32),
        scratch_types=[
            pltpu.VMEM((N,), jnp.int32),
            pltpu.VMEM((rows_w,), jnp.int32),
            pltpu.VMEM((rows_w, 64), jnp.float32),
            pltpu.VMEM((rows_w, PW), jnp.float32),
            pltpu.VMEM((rows_w,), jnp.int32),
            pltpu.VMEM((G, PW), jnp.float32),
            pltpu.VMEM_SHARED((G, PW), jnp.float32),
            pltpu.SemaphoreType.DMA,
        ],
    )
    def k(x3_hbm, nts_hbm, stg_hbm, out_hbm,
          nts_v, cidx_v, rows_v, prows_v, stg_v, gbuf_v, gsum_sh, sem):
        cid = lax.axis_index("c")
        sid = lax.axis_index("s")
        wid = sid * 2 + cid

        zz = jnp.zeros((16,), jnp.float32)
        for r in range(G):
            for cc in range(PW // 16):
                gbuf_v[r, pl.ds(cc * 16, 16)] = zz

        @pl.when(sid == 0)
        def _():
            pltpu.sync_copy(gbuf_v, gsum_sh)

        plsc.subcore_barrier()

        @pl.when(wid < n_active)
        def _():
            pltpu.sync_copy(nts_hbm, nts_v)
            base = wid * rows_w
            lane = lax.iota(jnp.int32, 16)
            for j in range(rows_w // 16):
                s = base + j * 16 + lane
                lo = jnp.full((16,), -1, jnp.int32)
                hi = jnp.full((16,), N, jnp.int32)
                for _step in range(14):
                    mid = jnp.maximum((lo + hi) >> 1, 0)
                    v = plsc.load_gather(nts_v, [mid])
                    pred = v < s
                    lo = jnp.where(pred, mid, lo)
                    hi = jnp.where(pred, hi, mid)
                cidx_v[pl.ds(j * 16, 16)] = jnp.minimum(hi, N - 1)
            pltpu.async_copy(x3_hbm.at[cidx_v], rows_v, sem).wait()
            e0 = jnp.where(lane == 0, 1.0, 0.0).astype(jnp.float32)
            for j in range(rows_w):
                for cc in range(4):
                    prows_v[j, pl.ds(cc * 16, 16)] = rows_v[j, pl.ds(cc * 16, 16)]
                prows_v[j, pl.ds(64, 16)] = e0
            pltpu.sync_copy(stg_hbm.at[pl.ds(base, rows_w)], stg_v)
            pltpu.sync_copy(prows_v, gsum_sh.at[stg_v], add=True)

        plsc.subcore_barrier()

        @pl.when(sid == 0)
        def _():
            pltpu.sync_copy(gsum_sh, gbuf_v)
            pltpu.sync_copy(gbuf_v, out_hbm.at[cid])

    return k(x3, nts, stg_p)


def _tc_mlp(p0, p1, fc1_w, fc1_b, fc2_w, fc2_b, fc3_w, fc3_b):
    def body(p0_ref, p1_ref, w1_ref, b1_ref, w2_ref, b2_ref, w3_ref, b3_ref,
             out_ref):
        ps = p0_ref[...] + p1_ref[...]
        cnt = jnp.maximum(ps[:, 64:65], 1.0)
        xg = ps[:, 0:64] / cnt
        v = jnp.dot(xg, w1_ref[...], preferred_element_type=jnp.float32) \
            + b1_ref[...]
        v = jnp.where(v > 0, v, jnp.exp(jnp.minimum(v, 0.0)) - 1.0)
        v = jnp.dot(v, w2_ref[...], preferred_element_type=jnp.float32) \
            + b2_ref[...]
        v = jnp.where(v > 0, v, jnp.exp(jnp.minimum(v, 0.0)) - 1.0)
        out_ref[...] = jnp.dot(v, w3_ref[...],
                               preferred_element_type=jnp.float32) + b3_ref[...]

    return pl.pallas_call(
        body,
        out_shape=jax.ShapeDtypeStruct((G, 1), jnp.float32),
    )(p0, p1, fc1_w, fc1_b.reshape(1, -1), fc2_w, fc2_b.reshape(1, -1),
      fc3_w, fc3_b.reshape(1, -1))


def kernel(x, edge_index, edge_attr, node_to_subgraph, subgraph_to_graph,
           W1_1, b1_1, W2_1, b2_1, root_1, bias_1,
           W1_2, b1_2, W2_2, b2_2, root_2, bias_2,
           W1_3, b1_3, W2_3, b2_3, root_3, bias_3,
           fc1_w, fc1_b, fc2_w, fc2_b, fc3_w, fc3_b):
    src = edge_index[0]
    dst = edge_index[1]
    ea_p = jnp.pad(edge_attr, ((0, 0), (0, 3)))

    conv = [
        (W1_1, b1_1, W2_1, b2_1, root_1, bias_1, 128, 32),
        (W1_2, b1_2, W2_2, b2_2, root_2, bias_2, 32, 64),
        (W1_3, b1_3, W2_3, b2_3, root_3, bias_3, 64, 64),
    ]
    for (w1, b1, w2, b2, root, bias, m_in, m_out) in conv:
        w1p = jnp.pad(w1, ((0, 3), (0, 0)))
        w2t = (w2.reshape(128, m_in, m_out).transpose(0, 2, 1)
               .reshape(128, m_out * m_in).astype(jnp.bfloat16))
        b2t = b2.reshape(m_in, m_out).T.reshape(1, -1)
        red = jnp.kron(jnp.eye(m_out, dtype=jnp.float32),
                       jnp.ones((m_in, 1), dtype=jnp.float32))
        x_src = _sc_gather(x, src, m_in)
        msg = _tc_msg(ea_p, x_src, w1p, b1.reshape(1, -1), w2t,
                      b2t, red, m_in, m_out)
        aggr_p = _sc_scatter(msg, dst, m_out)
        aggr0 = jnp.concatenate([aggr_p[0, p] for p in range(m_out // 32)],
                                axis=1)
        aggr1 = jnp.concatenate([aggr_p[1, p] for p in range(m_out // 32)],
                                axis=1)
        x = _tc_node(x, root, aggr0, aggr1, bias.reshape(1, -1),
                     m_in, m_out)

    pool_p = _sc_pool(x, node_to_subgraph.astype(jnp.int32),
                      subgraph_to_graph.astype(jnp.int32))
    out = _tc_mlp(pool_p[0], pool_p[1], fc1_w, fc1_b, fc2_w, fc2_b,
                  fc3_w, fc3_b)
    return out.reshape(-1)


# speedup vs baseline: 3.2483x; 1.0760x over previous
import functools

import jax
import jax.numpy as jnp
from jax import lax
from jax.experimental import pallas as pl
from jax.experimental.pallas import tpu as pltpu
from jax.experimental.pallas import tpu_sc as plsc

N = 10000
E = 160000
S = 2000
G = 64
NW = 32
PW = 80

_MESH = dict(core_axis_name="c", subcore_axis_name="s")


def _sc_gather(table, idx, m_in):
    per_w = E // NW
    chunk = 5000 if m_in <= 32 else 1000

    mesh = plsc.VectorSubcoreMesh(**_MESH)

    @functools.partial(
        pl.kernel, mesh=mesh,
        compiler_params=pltpu.CompilerParams(use_tc_tiling_on_sc=False, needs_layout_passes=False),
        out_type=jax.ShapeDtypeStruct((E, m_in), jnp.bfloat16),
        scratch_types=[
            pltpu.VMEM((chunk,), jnp.int32),
            pltpu.VMEM((chunk, m_in), jnp.bfloat16),
            pltpu.SemaphoreType.DMA,
        ],
    )
    def k(table_hbm, idx_hbm, out_hbm, idx_v, rows_v, sem):
        wid = lax.axis_index("s") * 2 + lax.axis_index("c")
        base = wid * per_w

        def body(i, carry):
            off = base + i * chunk
            pltpu.sync_copy(idx_hbm.at[pl.ds(off, chunk)], idx_v)
            pltpu.async_copy(table_hbm.at[idx_v], rows_v, sem).wait()
            pltpu.sync_copy(rows_v, out_hbm.at[pl.ds(off, chunk)])
            return carry

        lax.fori_loop(0, per_w // chunk, body, 0)

    return k(table, idx)


def _sc_scatter(msg, dst, m_out):
    per_w = E // NW
    chunk = 1000
    n_pad = 10240
    rows_s = n_pad // 16
    cw = 32
    phases = m_out // cw

    mesh = plsc.VectorSubcoreMesh(**_MESH)

    @functools.partial(
        pl.kernel, mesh=mesh,
        compiler_params=pltpu.CompilerParams(use_tc_tiling_on_sc=False, needs_layout_passes=False),
        out_type=jax.ShapeDtypeStruct((2, phases, n_pad, cw), jnp.float32),
        scratch_types=[
            pltpu.VMEM((chunk,), jnp.int32),
            pltpu.VMEM((chunk, cw), jnp.float32),
            pltpu.VMEM((rows_s, cw), jnp.float32),
            pltpu.VMEM_SHARED((n_pad, cw), jnp.float32),
        ],
    )
    def k(msg_hbm, dst_hbm, out_hbm, idx_v, buf_v, stage_v, aggr_sh):
        cid = lax.axis_index("c")
        sid = lax.axis_index("s")
        wid = sid * 2 + cid
        base = wid * per_w
        zz = jnp.zeros((16,), jnp.float32)

        def zrow(r, carry):
            for cc in range(cw // 16):
                stage_v[r, pl.ds(cc * 16, 16)] = zz
            return carry

        lax.fori_loop(0, rows_s, zrow, 0)

        for p in range(phases):
            pltpu.sync_copy(stage_v, aggr_sh.at[pl.ds(sid * rows_s, rows_s)])
            plsc.subcore_barrier()

            def body(i, carry):
                off = base + i * chunk
                pltpu.sync_copy(dst_hbm.at[pl.ds(off, chunk)], idx_v)
                pltpu.sync_copy(msg_hbm.at[p, pl.ds(off, chunk)], buf_v)
                pltpu.sync_copy(buf_v, aggr_sh.at[idx_v], add=True)
                return carry

            lax.fori_loop(0, per_w // chunk, body, 0)
            plsc.subcore_barrier()

            pltpu.sync_copy(aggr_sh.at[pl.ds(sid * rows_s, rows_s)], stage_v)
            pltpu.sync_copy(
                stage_v,
                out_hbm.at[cid, p, pl.ds(sid * rows_s, rows_s)])
            if p + 1 < phases:
                plsc.subcore_barrier()
                lax.fori_loop(0, rows_s, zrow, 0)

    return k(msg, dst)


def _tc_msg(ea_p, x_src, w1p, b1, w2t, b2t, red, m_in, m_out):
    blk = 640
    grid = E // blk
    cols = m_in * m_out
    rep = 128 // m_in

    def body(ea_ref, xs_ref, w1_ref, b1_ref, w2_ref, b2_ref, red_ref, out_ref):
        h = jnp.maximum(
            jnp.dot(ea_ref[...], w1_ref[...],
                    preferred_element_type=jnp.float32) + b1_ref[...], 0.0)
        w = jnp.dot(h.astype(jnp.bfloat16), w2_ref[...],
                    preferred_element_type=jnp.float32) + b2_ref[...]
        xs = xs_ref[...]
        xt128 = jnp.concatenate([xs] * rep, axis=1) if rep > 1 else xs
        xt = jnp.concatenate([xt128] * (cols // 128), axis=1)
        p = w * xt
        msg = jnp.dot(p, red_ref[...], preferred_element_type=jnp.float32)
        for q in range(m_out // 32):
            out_ref[q, :, :] = msg[:, q * 32:(q + 1) * 32]

    return pl.pallas_call(
        body,
        grid=(grid,),
        in_specs=[
            pl.BlockSpec((blk, 8), lambda i: (i, 0)),
            pl.BlockSpec((blk, m_in), lambda i: (i, 0)),
            pl.BlockSpec((8, 128), lambda i: (0, 0)),
            pl.BlockSpec((1, 128), lambda i: (0, 0)),
            pl.BlockSpec((128, cols), lambda i: (0, 0)),
            pl.BlockSpec((1, cols), lambda i: (0, 0)),
            pl.BlockSpec((cols, m_out), lambda i: (0, 0)),
        ],
        out_specs=pl.BlockSpec((m_out // 32, blk, 32), lambda i: (0, i, 0)),
        out_shape=jax.ShapeDtypeStruct((m_out // 32, E, 32), jnp.float32),
        compiler_params=pltpu.CompilerParams(
            dimension_semantics=("arbitrary",)),
    )(ea_p, x_src, w1p, b1, w2t, b2t, red)


def _tc_node(x, root, a0, a1, bias, m_in, m_out):
    blk = 400
    grid = N // blk

    def body(x_ref, r_ref, a0_ref, a1_ref, b_ref, out_ref, outb_ref):
        v = (jnp.dot(x_ref[...], r_ref[...],
                     preferred_element_type=jnp.float32)
             + a0_ref[...] + a1_ref[...] + b_ref[...])
        e = jnp.where(v > 0, v, jnp.exp(jnp.minimum(v, 0.0)) - 1.0)
        out_ref[...] = e
        outb_ref[...] = e.astype(jnp.bfloat16)

    return pl.pallas_call(
        body,
        grid=(grid,),
        in_specs=[
            pl.BlockSpec((blk, m_in), lambda i: (i, 0)),
            pl.BlockSpec((m_in, m_out), lambda i: (0, 0)),
            pl.BlockSpec((blk, m_out), lambda i: (i, 0)),
            pl.BlockSpec((blk, m_out), lambda i: (i, 0)),
            pl.BlockSpec((1, m_out), lambda i: (0, 0)),
        ],
        out_specs=[pl.BlockSpec((blk, m_out), lambda i: (i, 0)),
                   pl.BlockSpec((blk, m_out), lambda i: (i, 0))],
        out_shape=[jax.ShapeDtypeStruct((N, m_out), jnp.float32),
                   jax.ShapeDtypeStruct((N, m_out), jnp.bfloat16)],
        compiler_params=pltpu.CompilerParams(
            dimension_semantics=("arbitrary",)),
    )(x, root, a0, a1, bias)


def _sc_pool(x3, nts, stg_p):
    n_active = 25
    rows_w = S // n_active

    mesh = plsc.VectorSubcoreMesh(**_MESH)

    @functools.partial(
        pl.kernel, mesh=mesh,
        compiler_params=pltpu.CompilerParams(use_tc_tiling_on_sc=False, needs_layout_passes=False),
        out_type=jax.ShapeDtypeStruct((2, G, PW), jnp.float32),
        scratch_types=[
            pltpu.VMEM((N,), jnp.int32),
            pltpu.VMEM((rows_w,), jnp.int32),
            pltpu.VMEM((rows_w, 64), jnp.float32),
            pltpu.VMEM((rows_w, PW), jnp.float32),
            pltpu.VMEM((rows_w,), jnp.int32),
            pltpu.VMEM((G, PW), jnp.float32),
            pltpu.VMEM_SHARED((G, PW), jnp.float32),
            pltpu.SemaphoreType.DMA,
        ],
    )
    def k(x3_hbm, nts_hbm, stg_hbm, out_hbm,
          nts_v, cidx_v, rows_v, prows_v, stg_v, gbuf_v, gsum_sh, sem):
        cid = lax.axis_index("c")
        sid = lax.axis_index("s")
        wid = sid * 2 + cid

        zz = jnp.zeros((16,), jnp.float32)
        for r in range(G):
            for cc in range(PW // 16):
                gbuf_v[r, pl.ds(cc * 16, 16)] = zz

        @pl.when(sid == 0)
        def _():
            pltpu.sync_copy(gbuf_v, gsum_sh)

        plsc.subcore_barrier()

        @pl.when(wid < n_active)
        def _():
            pltpu.sync_copy(nts_hbm, nts_v)
            base = wid * rows_w
            lane = lax.iota(jnp.int32, 16)
            for j in range(rows_w // 16):
                s = base + j * 16 + lane
                lo = jnp.full((16,), -1, jnp.int32)
                hi = jnp.full((16,), N, jnp.int32)
                for _step in range(14):
                    mid = jnp.maximum((lo + hi) >> 1, 0)
                    v = plsc.load_gather(nts_v, [mid])
                    pred = v < s
                    lo = jnp.where(pred, mid, lo)
                    hi = jnp.where(pred, hi, mid)
                cidx_v[pl.ds(j * 16, 16)] = jnp.minimum(hi, N - 1)
            pltpu.async_copy(x3_hbm.at[cidx_v], rows_v, sem).wait()
            e0 = jnp.where(lane == 0, 1.0, 0.0).astype(jnp.float32)
            for j in range(rows_w):
                for cc in range(4):
                    prows_v[j, pl.ds(cc * 16, 16)] = rows_v[j, pl.ds(cc * 16, 16)]
                prows_v[j, pl.ds(64, 16)] = e0
            pltpu.sync_copy(stg_hbm.at[pl.ds(base, rows_w)], stg_v)
            pltpu.sync_copy(prows_v, gsum_sh.at[stg_v], add=True)

        plsc.subcore_barrier()

        @pl.when(sid == 0)
        def _():
            pltpu.sync_copy(gsum_sh, gbuf_v)
            pltpu.sync_copy(gbuf_v, out_hbm.at[cid])

    return k(x3, nts, stg_p)


def _tc_mlp(p0, p1, fc1_w, fc1_b, fc2_w, fc2_b, fc3_w, fc3_b):
    def body(p0_ref, p1_ref, w1_ref, b1_ref, w2_ref, b2_ref, w3_ref, b3_ref,
             out_ref):
        ps = p0_ref[...] + p1_ref[...]
        cnt = jnp.maximum(ps[:, 64:65], 1.0)
        xg = ps[:, 0:64] / cnt
        v = jnp.dot(xg, w1_ref[...], preferred_element_type=jnp.float32) \
            + b1_ref[...]
        v = jnp.where(v > 0, v, jnp.exp(jnp.minimum(v, 0.0)) - 1.0)
        v = jnp.dot(v, w2_ref[...], preferred_element_type=jnp.float32) \
            + b2_ref[...]
        v = jnp.where(v > 0, v, jnp.exp(jnp.minimum(v, 0.0)) - 1.0)
        out_ref[...] = jnp.dot(v, w3_ref[...],
                               preferred_element_type=jnp.float32) + b3_ref[...]

    return pl.pallas_call(
        body,
        out_shape=jax.ShapeDtypeStruct((G, 1), jnp.float32),
    )(p0, p1, fc1_w, fc1_b.reshape(1, -1), fc2_w, fc2_b.reshape(1, -1),
      fc3_w, fc3_b.reshape(1, -1))


def kernel(x, edge_index, edge_attr, node_to_subgraph, subgraph_to_graph,
           W1_1, b1_1, W2_1, b2_1, root_1, bias_1,
           W1_2, b1_2, W2_2, b2_2, root_2, bias_2,
           W1_3, b1_3, W2_3, b2_3, root_3, bias_3,
           fc1_w, fc1_b, fc2_w, fc2_b, fc3_w, fc3_b):
    src = edge_index[0]
    dst = edge_index[1]
    ea_p = jnp.pad(edge_attr, ((0, 0), (0, 3)))

    conv = [
        (W1_1, b1_1, W2_1, b2_1, root_1, bias_1, 128, 32),
        (W1_2, b1_2, W2_2, b2_2, root_2, bias_2, 32, 64),
        (W1_3, b1_3, W2_3, b2_3, root_3, bias_3, 64, 64),
    ]
    xb = x.astype(jnp.bfloat16)
    for (w1, b1, w2, b2, root, bias, m_in, m_out) in conv:
        w1p = jnp.pad(w1, ((0, 3), (0, 0)))
        w2t = (w2.reshape(128, m_in, m_out).transpose(0, 2, 1)
               .reshape(128, m_out * m_in).astype(jnp.bfloat16))
        b2t = b2.reshape(m_in, m_out).T.reshape(1, -1)
        red = jnp.kron(jnp.eye(m_out, dtype=jnp.float32),
                       jnp.ones((m_in, 1), dtype=jnp.float32))
        x_src = _sc_gather(xb, src, m_in)
        msg = _tc_msg(ea_p, x_src, w1p, b1.reshape(1, -1), w2t,
                      b2t, red, m_in, m_out)
        aggr_p = _sc_scatter(msg, dst, m_out)
        aggr0 = jnp.concatenate([aggr_p[0, p] for p in range(m_out // 32)],
                                axis=1)
        aggr1 = jnp.concatenate([aggr_p[1, p] for p in range(m_out // 32)],
                                axis=1)
        x, xb = _tc_node(x, root, aggr0, aggr1, bias.reshape(1, -1),
                         m_in, m_out)

    pool_p = _sc_pool(x, node_to_subgraph.astype(jnp.int32),
                      subgraph_to_graph.astype(jnp.int32))
    out = _tc_mlp(pool_p[0], pool_p[1], fc1_w, fc1_b, fc2_w, fc2_b,
                  fc3_w, fc3_b)
    return out.reshape(-1)
